# baseline (speedup 1.0000x reference)
import jax
import jax.numpy as jnp
from jax import lax
from jax.experimental import pallas as pl
from jax.experimental.pallas import tpu as pltpu
from jax.experimental.pallas import tpu_sc as plsc

N = 50000
D = 64
HW = 32
NT = 16
NC = 2
C = 128
RPT = 3200
NROW = NT * RPT
TRASH = NROW - 1

E = 800000
CJ_AGG = 392
E_AGG = NT * CJ_AGG * C
CJ_DEG = 196
E_DEG = NC * NT * CJ_DEG * C
L = 100000
CJ_DEC = 26
L_PAD = NC * NT * CJ_DEC * C

GRP = 56
BR = 2000
BRL = 2048

_mesh = plsc.VectorSubcoreMesh(
    core_axis_name="c", subcore_axis_name="s", num_cores=NC, num_subcores=NT
)
_sc_params = pltpu.CompilerParams(use_tc_tiling_on_sc=False)


def _deg_body(dst_hbm, ones_hbm, zeros_hbm, out_hbm, idx_all, ones_v, buf_v, acc, sem):
    c = lax.axis_index("c")
    s = lax.axis_index("s")
    base = s * RPT
    pltpu.sync_copy(zeros_hbm, buf_v)
    pltpu.sync_copy(ones_hbm, ones_v)
    pltpu.sync_copy(dst_hbm.at[c * NT + s], idx_all)

    def zero_step(k, _):
        pltpu.sync_copy(buf_v, acc.at[pl.ds(base + k * C, C)])
        return 0

    lax.fori_loop(0, RPT // C, zero_step, 0)
    plsc.subcore_barrier()

    K = 7
    def step(t, _):
        for b in range(K):
            pltpu.async_copy(ones_v, acc.at[idx_all.at[t * K + b]], sem, add=True)
        for b in range(K):
            pltpu.make_async_copy(ones_v, acc.at[idx_all.at[t * K + b]], sem).wait()
        return 0

    lax.fori_loop(0, CJ_DEG // K, step, 0)
    plsc.subcore_barrier()

    def out_step(k, _):
        pltpu.sync_copy(acc.at[pl.ds(base + k * C, C)], buf_v)
        pltpu.sync_copy(buf_v, out_hbm.at[pl.ds(c * NROW + base + k * C, C)])
        return 0

    lax.fori_loop(0, RPT // C, out_step, 0)


def _deg_call(dstd, ones16, zeros16):
    return pl.kernel(
        _deg_body,
        out_type=jax.ShapeDtypeStruct((NC * NROW, 16), jnp.float32),
        mesh=_mesh,
        compiler_params=_sc_params,
        scratch_types=[
            pltpu.VMEM((CJ_DEG, C), jnp.int32),
            pltpu.VMEM((C, 16), jnp.float32),
            pltpu.VMEM((C, 16), jnp.float32),
            pltpu.VMEM_SHARED((NROW, 16), jnp.float32),
            pltpu.SemaphoreType.DMA,
        ],
    )(dstd, ones16, zeros16)


def _agg_body(ga, gb, src_hbm, dst_hbm, zeros_hbm, out_hbm,
              sidx_all, didx_all, rows_a, rows_b, buf, acc, sem_a, sem_b):
    c = lax.axis_index("c")
    s = lax.axis_index("s")
    base = s * RPT
    pltpu.sync_copy(zeros_hbm, buf)

    def zero_step(k, _):
        pltpu.sync_copy(buf, acc.at[pl.ds(base + k * C, C)])
        return 0

    lax.fori_loop(0, RPT // C, zero_step, 0)
    plsc.subcore_barrier()

    def run(g_hbm):
        def group(g, _):
            pltpu.sync_copy(src_hbm.at[s, pl.ds(g * GRP, GRP)], sidx_all)
            pltpu.sync_copy(dst_hbm.at[s, pl.ds(g * GRP, GRP)], didx_all)
            pltpu.async_copy(g_hbm.at[sidx_all.at[0]], rows_a, sem_a)

            def step(t, _):
                ja = 2 * t
                jb = 2 * t + 1
                pltpu.async_copy(g_hbm.at[sidx_all.at[jb]], rows_b, sem_b)
                pltpu.make_async_copy(g_hbm.at[sidx_all.at[ja]], rows_a, sem_a).wait()
                pltpu.sync_copy(rows_a, acc.at[didx_all.at[ja]], add=True)

                @pl.when(t < GRP // 2 - 1)
                def _():
                    pltpu.async_copy(g_hbm.at[sidx_all.at[ja + 2]], rows_a, sem_a)

                pltpu.make_async_copy(g_hbm.at[sidx_all.at[jb]], rows_b, sem_b).wait()
                pltpu.sync_copy(rows_b, acc.at[didx_all.at[jb]], add=True)
                return 0

            lax.fori_loop(0, GRP // 2, step, 0)
            return 0

        lax.fori_loop(0, CJ_AGG // GRP, group, 0)

    pl.when(c == 0)(lambda: run(ga))
    pl.when(c == 1)(lambda: run(gb))
    plsc.subcore_barrier()

    def out_step(k, _):
        pltpu.sync_copy(acc.at[pl.ds(base + k * C, C)], rows_a)
        pltpu.sync_copy(rows_a, out_hbm.at[pl.ds(c * NROW + base + k * C, C)])
        return 0

    lax.fori_loop(0, RPT // C, out_step, 0)


def _agg_call(ga, gb, srcp, dstp, zeros32):
    return pl.kernel(
        _agg_body,
        out_type=jax.ShapeDtypeStruct((NC * NROW, HW), jnp.float32),
        mesh=_mesh,
        compiler_params=_sc_params,
        scratch_types=[
            pltpu.VMEM((GRP, C), jnp.int32),
            pltpu.VMEM((GRP, C), jnp.int32),
            pltpu.VMEM((C, HW), jnp.float32),
            pltpu.VMEM((C, HW), jnp.float32),
            pltpu.VMEM((C, HW), jnp.float32),
            pltpu.VMEM_SHARED((NROW, HW), jnp.float32),
            pltpu.SemaphoreType.DMA,
            pltpu.SemaphoreType.DMA,
        ],
    )(ga, gb, srcp, dstp, zeros32)


def _dec_body(z_hbm, es_hbm, ed_hbm, sout_hbm, dout_hbm,
              sidx_all, didx_all, srows_a, drows_a, srows_b, drows_b,
              sem_sa, sem_da, sem_sb, sem_db):
    c = lax.axis_index("c")
    s = lax.axis_index("s")
    w = c * NT + s
    pltpu.sync_copy(es_hbm.at[w], sidx_all)
    pltpu.sync_copy(ed_hbm.at[w], didx_all)
    pltpu.async_copy(z_hbm.at[sidx_all.at[0]], srows_a, sem_sa)
    pltpu.async_copy(z_hbm.at[didx_all.at[0]], drows_a, sem_da)

    def step(t, _):
        ja = 2 * t
        jb = 2 * t + 1
        offa = (w * CJ_DEC + ja) * C
        offb = (w * CJ_DEC + jb) * C
        pltpu.async_copy(z_hbm.at[sidx_all.at[jb]], srows_b, sem_sb)
        pltpu.async_copy(z_hbm.at[didx_all.at[jb]], drows_b, sem_db)
        pltpu.make_async_copy(z_hbm.at[sidx_all.at[ja]], srows_a, sem_sa).wait()
        pltpu.sync_copy(srows_a, sout_hbm.at[pl.ds(offa, C)])
        pltpu.make_async_copy(z_hbm.at[didx_all.at[ja]], drows_a, sem_da).wait()
        pltpu.sync_copy(drows_a, dout_hbm.at[pl.ds(offa, C)])

        @pl.when(t < CJ_DEC // 2 - 1)
        def _():
            pltpu.async_copy(z_hbm.at[sidx_all.at[ja + 2]], srows_a, sem_sa)
            pltpu.async_copy(z_hbm.at[didx_all.at[ja + 2]], drows_a, sem_da)

        pltpu.make_async_copy(z_hbm.at[sidx_all.at[jb]], srows_b, sem_sb).wait()
        pltpu.sync_copy(srows_b, sout_hbm.at[pl.ds(offb, C)])
        pltpu.make_async_copy(z_hbm.at[didx_all.at[jb]], drows_b, sem_db).wait()
        pltpu.sync_copy(drows_b, dout_hbm.at[pl.ds(offb, C)])
        return 0

    lax.fori_loop(0, CJ_DEC // 2, step, 0)


def _dec_call(z2, es, ed):
    return pl.kernel(
        _dec_body,
        out_type=(
            jax.ShapeDtypeStruct((L_PAD, D), jnp.float32),
            jax.ShapeDtypeStruct((L_PAD, D), jnp.float32),
        ),
        mesh=_mesh,
        compiler_params=_sc_params,
        scratch_types=[
            pltpu.VMEM((CJ_DEC, C), jnp.int32),
            pltpu.VMEM((CJ_DEC, C), jnp.int32),
            pltpu.VMEM((C, D), jnp.float32),
            pltpu.VMEM((C, D), jnp.float32),
            pltpu.VMEM((C, D), jnp.float32),
            pltpu.VMEM((C, D), jnp.float32),
            pltpu.SemaphoreType.DMA,
            pltpu.SemaphoreType.DMA,
            pltpu.SemaphoreType.DMA,
            pltpu.SemaphoreType.DMA,
        ],
    )(z2, es, ed)


def _dinv_block(deg_ref):
    dsum = deg_ref[0][:, 0:1] + deg_ref[1][:, 0:1]
    return lax.rsqrt(dsum + 1.0)


def _prep1_body(deg_ref, emb_ref, w1_ref, outg_ref, outh_ref):
    dinv = _dinv_block(deg_ref)
    h = jnp.dot(emb_ref[...], w1_ref[...], preferred_element_type=jnp.float32)
    outh_ref[...] = h
    g = h * dinv
    outg_ref[0] = g[:, :HW]
    outg_ref[1] = g[:, HW:]


def _mid_body(deg_ref, acc_ref, h_ref, b_ref, w2_ref, outg_ref, outh_ref):
    dinv = _dinv_block(deg_ref)
    accf = jnp.concatenate([acc_ref[0], acc_ref[1]], axis=1)
    z = jnp.maximum(accf * dinv + h_ref[...] * (dinv * dinv) + b_ref[...], 0.0)
    h2 = jnp.dot(z, w2_ref[...], preferred_element_type=jnp.float32)
    outh_ref[...] = h2
    g2 = h2 * dinv
    outg_ref[0] = g2[:, :HW]
    outg_ref[1] = g2[:, HW:]


def _fin_body(deg_ref, acc_ref, h_ref, b_ref, out_ref):
    dinv = _dinv_block(deg_ref)
    accf = jnp.concatenate([acc_ref[0], acc_ref[1]], axis=1)
    out_ref[...] = accf * dinv + h_ref[...] * (dinv * dinv) + b_ref[...]


def _dot_body(s_ref, d_ref, o_ref):
    o_ref[...] = jnp.sum(s_ref[...] * d_ref[...], axis=1).reshape(BRL // 256, 256)


_deg_spec = pl.BlockSpec((NC, BR, 16), lambda i: (0, i, 0))
_acc_spec = pl.BlockSpec((NC, BR, HW), lambda i: (0, i, 0))
_row_spec = pl.BlockSpec((BR, D), lambda i: (i, 0))
_g_spec = pl.BlockSpec((NC, BR, HW), lambda i: (0, i, 0))
_w_spec = pl.BlockSpec((D, D), lambda i: (0, 0))
_b_spec = pl.BlockSpec((1, D), lambda i: (0, 0))


def _prep1(deg3, emb, W1):
    return pl.pallas_call(
        _prep1_body,
        grid=(N // BR,),
        in_specs=[_deg_spec, _row_spec, _w_spec],
        out_specs=(_g_spec, _row_spec),
        out_shape=(
            jax.ShapeDtypeStruct((NC, N, HW), jnp.float32),
            jax.ShapeDtypeStruct((N, D), jnp.float32),
        ),
    )(deg3, emb, W1)


def _mid(deg3, acc3, h1, b1, W2):
    return pl.pallas_call(
        _mid_body,
        grid=(N // BR,),
        in_specs=[_deg_spec, _acc_spec, _row_spec, _b_spec, _w_spec],
        out_specs=(_g_spec, _row_spec),
        out_shape=(
            jax.ShapeDtypeStruct((NC, N, HW), jnp.float32),
            jax.ShapeDtypeStruct((N, D), jnp.float32),
        ),
    )(deg3, acc3, h1, b1, W2)


def _fin(deg3, acc3, h2, b2):
    return pl.pallas_call(
        _fin_body,
        grid=(N // BR,),
        in_specs=[_deg_spec, _acc_spec, _row_spec, _b_spec],
        out_specs=_row_spec,
        out_shape=jax.ShapeDtypeStruct((N, D), jnp.float32),
    )(deg3, acc3, h2, b2)


def _dot(srows, drows):
    return pl.pallas_call(
        _dot_body,
        grid=(L_PAD // BRL,),
        in_specs=[
            pl.BlockSpec((BRL, D), lambda i: (i, 0)),
            pl.BlockSpec((BRL, D), lambda i: (i, 0)),
        ],
        out_specs=pl.BlockSpec((BRL // 256, 256), lambda i: (i, 0)),
        out_shape=jax.ShapeDtypeStruct((L_PAD // 256, 256), jnp.float32),
    )(srows, drows)


def kernel(edge_index, edge_label_index, emb, W1, b1, W2, b2):
    src = edge_index[0]
    dst = edge_index[1]

    ones16 = jnp.ones((C, 16), jnp.float32)
    zeros16 = jnp.zeros((C, 16), jnp.float32)
    zeros32 = jnp.zeros((C, HW), jnp.float32)

    dstd = jnp.concatenate(
        [dst, jnp.full((E_DEG - E,), TRASH, jnp.int32)]
    ).reshape(NC * NT, CJ_DEG, C)
    deg3 = _deg_call(dstd, ones16, zeros16).reshape(NC, NROW, 16)

    g1, h1 = _prep1(deg3, emb, W1)

    srcp = jnp.concatenate(
        [src, jnp.zeros((E_AGG - E,), jnp.int32)]
    ).reshape(NT, CJ_AGG, C)
    dstp = jnp.concatenate(
        [dst, jnp.full((E_AGG - E,), TRASH, jnp.int32)]
    ).reshape(NT, CJ_AGG, C)

    acc1 = _agg_call(g1[0], g1[1], srcp, dstp, zeros32).reshape(NC, NROW, HW)
    g2, h2 = _mid(deg3, acc1, h1, b1.reshape(1, D), W2)
    acc2 = _agg_call(g2[0], g2[1], srcp, dstp, zeros32).reshape(NC, NROW, HW)
    z2 = _fin(deg3, acc2, h2, b2.reshape(1, D))

    es = jnp.concatenate(
        [edge_label_index[0], jnp.zeros((L_PAD - L,), jnp.int32)]
    ).reshape(NC * NT, CJ_DEC, C)
    ed = jnp.concatenate(
        [edge_label_index[1], jnp.zeros((L_PAD - L,), jnp.int32)]
    ).reshape(NC * NT, CJ_DEC, C)
    srows, drows = _dec_call(z2, es, ed)
    return _dot(srows, drows).reshape(L_PAD)[:L]


# speedup vs baseline: 1.2304x; 1.2304x over previous
import jax
import jax.numpy as jnp
from jax import lax
from jax.experimental import pallas as pl
from jax.experimental.pallas import tpu as pltpu
from jax.experimental.pallas import tpu_sc as plsc

N = 50000
D = 64
HW = 32
NT = 16
NC = 2
C = 128
RPT = 3200
NROW = NT * RPT
TRASH = NROW - 1

E = 800000
CJ_AGG = 392
E_AGG = NT * CJ_AGG * C
CJ_DEG = 196
E_DEG = NC * NT * CJ_DEG * C
L = 100000
CJ_DEC = 25
L_PAD = NC * NT * CJ_DEC * C

GRP = 56
BR = 2000
BRL = 2048

_mesh = plsc.VectorSubcoreMesh(
    core_axis_name="c", subcore_axis_name="s", num_cores=NC, num_subcores=NT
)
_sc_params = pltpu.CompilerParams(use_tc_tiling_on_sc=False)


def _deg_body(dst_hbm, ones_hbm, zeros_hbm, out_hbm, idx_all, ones_v, buf_v, acc, sem):
    c = lax.axis_index("c")
    s = lax.axis_index("s")
    base = s * RPT
    pltpu.sync_copy(zeros_hbm, buf_v)
    pltpu.sync_copy(ones_hbm, ones_v)
    pltpu.sync_copy(dst_hbm.at[c * NT + s], idx_all)

    def zero_step(k, _):
        pltpu.sync_copy(buf_v, acc.at[pl.ds(base + k * C, C)])
        return 0

    lax.fori_loop(0, RPT // C, zero_step, 0)
    plsc.subcore_barrier()

    K = 7
    def step(t, _):
        for b in range(K):
            pltpu.async_copy(ones_v, acc.at[idx_all.at[t * K + b]], sem, add=True)
        for b in range(K):
            pltpu.make_async_copy(ones_v, acc.at[idx_all.at[t * K + b]], sem).wait()
        return 0

    lax.fori_loop(0, CJ_DEG // K, step, 0)
    plsc.subcore_barrier()

    def out_step(k, _):
        pltpu.sync_copy(acc.at[pl.ds(base + k * C, C)], buf_v)
        pltpu.sync_copy(buf_v, out_hbm.at[pl.ds(c * NROW + base + k * C, C)])
        return 0

    lax.fori_loop(0, RPT // C, out_step, 0)


def _deg_call(dstd, ones16, zeros16):
    return pl.kernel(
        _deg_body,
        out_type=jax.ShapeDtypeStruct((NC * NROW, 16), jnp.float32),
        mesh=_mesh,
        compiler_params=_sc_params,
        scratch_types=[
            pltpu.VMEM((CJ_DEG, C), jnp.int32),
            pltpu.VMEM((C, 16), jnp.float32),
            pltpu.VMEM((C, 16), jnp.float32),
            pltpu.VMEM_SHARED((NROW, 16), jnp.float32),
            pltpu.SemaphoreType.DMA,
        ],
    )(dstd, ones16, zeros16)


def _agg_body(g3, src_hbm, dst_hbm, zeros_hbm, out_hbm,
              sidx_all, didx_all, rows_a, rows_b, buf, acc, sem_a, sem_b):
    c = lax.axis_index("c")
    s = lax.axis_index("s")
    base = s * RPT
    pltpu.sync_copy(zeros_hbm, buf)

    def zero_step(k, _):
        pltpu.sync_copy(buf, acc.at[pl.ds(base + k * C, C)])
        return 0

    lax.fori_loop(0, RPT // C, zero_step, 0)
    plsc.subcore_barrier()

    def run(g_hbm):
        def group(g, _):
            pltpu.sync_copy(src_hbm.at[s, pl.ds(g * GRP, GRP)], sidx_all)
            pltpu.sync_copy(dst_hbm.at[s, pl.ds(g * GRP, GRP)], didx_all)
            pltpu.async_copy(g_hbm.at[sidx_all.at[0]], rows_a, sem_a)

            def step(t, _):
                ja = 2 * t
                jb = 2 * t + 1
                pltpu.async_copy(g_hbm.at[sidx_all.at[jb]], rows_b, sem_b)
                pltpu.make_async_copy(g_hbm.at[sidx_all.at[ja]], rows_a, sem_a).wait()
                pltpu.sync_copy(rows_a, acc.at[didx_all.at[ja]], add=True)

                @pl.when(t < GRP // 2 - 1)
                def _():
                    pltpu.async_copy(g_hbm.at[sidx_all.at[ja + 2]], rows_a, sem_a)

                pltpu.make_async_copy(g_hbm.at[sidx_all.at[jb]], rows_b, sem_b).wait()
                pltpu.sync_copy(rows_b, acc.at[didx_all.at[jb]], add=True)
                return 0

            lax.fori_loop(0, GRP // 2, step, 0)
            return 0

        lax.fori_loop(0, CJ_AGG // GRP, group, 0)

    pl.when(c == 0)(lambda: run(g3.at[0]))
    pl.when(c == 1)(lambda: run(g3.at[1]))
    plsc.subcore_barrier()

    def out_step(k, _):
        pltpu.sync_copy(acc.at[pl.ds(base + k * C, C)], rows_a)
        pltpu.sync_copy(rows_a, out_hbm.at[pl.ds(c * NROW + base + k * C, C)])
        return 0

    lax.fori_loop(0, RPT // C, out_step, 0)


def _agg_call(g3, srcp, dstp, zeros32):
    return pl.kernel(
        _agg_body,
        out_type=jax.ShapeDtypeStruct((NC * NROW, HW), jnp.float32),
        mesh=_mesh,
        compiler_params=_sc_params,
        scratch_types=[
            pltpu.VMEM((GRP, C), jnp.int32),
            pltpu.VMEM((GRP, C), jnp.int32),
            pltpu.VMEM((C, HW), jnp.float32),
            pltpu.VMEM((C, HW), jnp.float32),
            pltpu.VMEM((C, HW), jnp.float32),
            pltpu.VMEM_SHARED((NROW, HW), jnp.float32),
            pltpu.SemaphoreType.DMA,
            pltpu.SemaphoreType.DMA,
        ],
    )(g3, srcp, dstp, zeros32)


def _dec_body(z_hbm, es_hbm, ed_hbm, sout_hbm, dout_hbm,
              sidx_all, didx_all, srows, drows, sem_s, sem_d):
    c = lax.axis_index("c")
    s = lax.axis_index("s")
    w = c * NT + s
    pltpu.sync_copy(es_hbm.at[w], sidx_all)
    pltpu.sync_copy(ed_hbm.at[w], didx_all)

    def step(j, _):
        off = (w * CJ_DEC + j) * C
        pltpu.async_copy(z_hbm.at[sidx_all.at[j]], srows, sem_s)
        pltpu.async_copy(z_hbm.at[didx_all.at[j]], drows, sem_d)
        pltpu.make_async_copy(z_hbm.at[sidx_all.at[j]], srows, sem_s).wait()
        pltpu.sync_copy(srows, sout_hbm.at[pl.ds(off, C)])
        pltpu.make_async_copy(z_hbm.at[didx_all.at[j]], drows, sem_d).wait()
        pltpu.sync_copy(drows, dout_hbm.at[pl.ds(off, C)])
        return 0

    lax.fori_loop(0, CJ_DEC, step, 0)


def _dec_call(z2, es, ed):
    return pl.kernel(
        _dec_body,
        out_type=(
            jax.ShapeDtypeStruct((L_PAD, D), jnp.float32),
            jax.ShapeDtypeStruct((L_PAD, D), jnp.float32),
        ),
        mesh=_mesh,
        compiler_params=_sc_params,
        scratch_types=[
            pltpu.VMEM((CJ_DEC, C), jnp.int32),
            pltpu.VMEM((CJ_DEC, C), jnp.int32),
            pltpu.VMEM((C, D), jnp.float32),
            pltpu.VMEM((C, D), jnp.float32),
            pltpu.SemaphoreType.DMA,
            pltpu.SemaphoreType.DMA,
        ],
    )(z2, es, ed)


def _dinv_block(deg_ref):
    dsum = deg_ref[0][:, 0:1] + deg_ref[1][:, 0:1]
    return lax.rsqrt(dsum + 1.0)


def _prep1_body(deg_ref, emb_ref, w1_ref, outg_ref, outh_ref):
    dinv = _dinv_block(deg_ref)
    h = jnp.dot(emb_ref[...], w1_ref[...], preferred_element_type=jnp.float32)
    outh_ref[...] = h
    g = h * dinv
    outg_ref[0] = g[:, :HW]
    outg_ref[1] = g[:, HW:]


def _mid_body(deg_ref, acc_ref, h_ref, b_ref, w2_ref, outg_ref, outh_ref):
    dinv = _dinv_block(deg_ref)
    accf = jnp.concatenate([acc_ref[0], acc_ref[1]], axis=1)
    z = jnp.maximum(accf * dinv + h_ref[...] * (dinv * dinv) + b_ref[...], 0.0)
    h2 = jnp.dot(z, w2_ref[...], preferred_element_type=jnp.float32)
    outh_ref[...] = h2
    g2 = h2 * dinv
    outg_ref[0] = g2[:, :HW]
    outg_ref[1] = g2[:, HW:]


def _fin_body(deg_ref, acc_ref, h_ref, b_ref, out_ref):
    dinv = _dinv_block(deg_ref)
    accf = jnp.concatenate([acc_ref[0], acc_ref[1]], axis=1)
    out_ref[...] = accf * dinv + h_ref[...] * (dinv * dinv) + b_ref[...]


def _dot_body(s_ref, d_ref, o_ref):
    o_ref[...] = jnp.sum(s_ref[...] * d_ref[...], axis=1).reshape(BRL // 256, 256)


_deg_spec = pl.BlockSpec((NC, BR, 16), lambda i: (0, i, 0))
_acc_spec = pl.BlockSpec((NC, BR, HW), lambda i: (0, i, 0))
_row_spec = pl.BlockSpec((BR, D), lambda i: (i, 0))
_g_spec = pl.BlockSpec((NC, BR, HW), lambda i: (0, i, 0))
_w_spec = pl.BlockSpec((D, D), lambda i: (0, 0))
_b_spec = pl.BlockSpec((1, D), lambda i: (0, 0))


def _prep1(deg3, emb, W1):
    return pl.pallas_call(
        _prep1_body,
        grid=(N // BR,),
        in_specs=[_deg_spec, _row_spec, _w_spec],
        out_specs=(_g_spec, _row_spec),
        out_shape=(
            jax.ShapeDtypeStruct((NC, N, HW), jnp.float32),
            jax.ShapeDtypeStruct((N, D), jnp.float32),
        ),
    )(deg3, emb, W1)


def _mid(deg3, acc3, h1, b1, W2):
    return pl.pallas_call(
        _mid_body,
        grid=(N // BR,),
        in_specs=[_deg_spec, _acc_spec, _row_spec, _b_spec, _w_spec],
        out_specs=(_g_spec, _row_spec),
        out_shape=(
            jax.ShapeDtypeStruct((NC, N, HW), jnp.float32),
            jax.ShapeDtypeStruct((N, D), jnp.float32),
        ),
    )(deg3, acc3, h1, b1, W2)


def _fin(deg3, acc3, h2, b2):
    return pl.pallas_call(
        _fin_body,
        grid=(N // BR,),
        in_specs=[_deg_spec, _acc_spec, _row_spec, _b_spec],
        out_specs=_row_spec,
        out_shape=jax.ShapeDtypeStruct((N, D), jnp.float32),
    )(deg3, acc3, h2, b2)


def _dot(srows, drows):
    return pl.pallas_call(
        _dot_body,
        grid=(L_PAD // BRL,),
        in_specs=[
            pl.BlockSpec((BRL, D), lambda i: (i, 0)),
            pl.BlockSpec((BRL, D), lambda i: (i, 0)),
        ],
        out_specs=pl.BlockSpec((BRL // 256, 256), lambda i: (i, 0)),
        out_shape=jax.ShapeDtypeStruct((L_PAD // 256, 256), jnp.float32),
    )(srows, drows)


def kernel(edge_index, edge_label_index, emb, W1, b1, W2, b2):
    src = edge_index[0]
    dst = edge_index[1]

    ones16 = jnp.ones((C, 16), jnp.float32)
    zeros16 = jnp.zeros((C, 16), jnp.float32)
    zeros32 = jnp.zeros((C, HW), jnp.float32)

    dstd = jnp.concatenate(
        [dst, jnp.full((E_DEG - E,), TRASH, jnp.int32)]
    ).reshape(NC * NT, CJ_DEG, C)
    deg3 = _deg_call(dstd, ones16, zeros16).reshape(NC, NROW, 16)

    g1, h1 = _prep1(deg3, emb, W1)

    srcp = jnp.concatenate(
        [src, jnp.zeros((E_AGG - E,), jnp.int32)]
    ).reshape(NT, CJ_AGG, C)
    dstp = jnp.concatenate(
        [dst, jnp.full((E_AGG - E,), TRASH, jnp.int32)]
    ).reshape(NT, CJ_AGG, C)

    acc1 = _agg_call(g1, srcp, dstp, zeros32).reshape(NC, NROW, HW)
    g2, h2 = _mid(deg3, acc1, h1, b1.reshape(1, D), W2)
    acc2 = _agg_call(g2, srcp, dstp, zeros32).reshape(NC, NROW, HW)
    z2 = _fin(deg3, acc2, h2, b2.reshape(1, D))

    es = jnp.concatenate(
        [edge_label_index[0], jnp.zeros((L_PAD - L,), jnp.int32)]
    ).reshape(NC * NT, CJ_DEC, C)
    ed = jnp.concatenate(
        [edge_label_index[1], jnp.zeros((L_PAD - L,), jnp.int32)]
    ).reshape(NC * NT, CJ_DEC, C)
    srows, drows = _dec_call(z2, es, ed)
    return _dot(srows, drows).reshape(L_PAD)[:L]


# speedup vs baseline: 1.3331x; 1.0835x over previous
import jax
import jax.numpy as jnp
from jax import lax
from jax.experimental import pallas as pl
from jax.experimental.pallas import tpu as pltpu
from jax.experimental.pallas import tpu_sc as plsc

N = 50000
D = 64
HW = 32
NT = 16
NC = 2
C = 128
RPT = 3200
NROW = NT * RPT
TRASH = NROW - 1

E = 800000
CJ_AGG = 392
E_AGG = NT * CJ_AGG * C
CJ_DEG = 196
E_DEG = NC * NT * CJ_DEG * C
L = 100000
CJ_DEC = 25
L_PAD = NC * NT * CJ_DEC * C

GRP = 28
BR = 2000
BRL = 2048

_mesh = plsc.VectorSubcoreMesh(
    core_axis_name="c", subcore_axis_name="s", num_cores=NC, num_subcores=NT
)
_sc_params = pltpu.CompilerParams(use_tc_tiling_on_sc=False)


def _deg_body(dst_hbm, ones_hbm, zeros_hbm, out_hbm, idx_all, ones_v, buf_v, acc, sem):
    c = lax.axis_index("c")
    s = lax.axis_index("s")
    base = s * RPT
    pltpu.sync_copy(zeros_hbm, buf_v)
    pltpu.sync_copy(ones_hbm, ones_v)
    pltpu.sync_copy(dst_hbm.at[c * NT + s], idx_all)

    def zero_step(k, _):
        pltpu.sync_copy(buf_v, acc.at[pl.ds(base + k * C, C)])
        return 0

    lax.fori_loop(0, RPT // C, zero_step, 0)
    plsc.subcore_barrier()

    K = 7
    def step(t, _):
        for b in range(K):
            pltpu.async_copy(ones_v, acc.at[idx_all.at[t * K + b]], sem, add=True)
        for b in range(K):
            pltpu.make_async_copy(ones_v, acc.at[idx_all.at[t * K + b]], sem).wait()
        return 0

    lax.fori_loop(0, CJ_DEG // K, step, 0)
    plsc.subcore_barrier()

    def out_step(k, _):
        pltpu.sync_copy(acc.at[pl.ds(base + k * C, C)], buf_v)
        pltpu.sync_copy(buf_v, out_hbm.at[pl.ds(c * NROW + base + k * C, C)])
        return 0

    lax.fori_loop(0, RPT // C, out_step, 0)


def _deg_call(dstd, ones16, zeros16):
    return pl.kernel(
        _deg_body,
        out_type=jax.ShapeDtypeStruct((NC * NROW, 16), jnp.float32),
        mesh=_mesh,
        compiler_params=_sc_params,
        scratch_types=[
            pltpu.VMEM((CJ_DEG, C), jnp.int32),
            pltpu.VMEM((C, 16), jnp.float32),
            pltpu.VMEM((C, 16), jnp.float32),
            pltpu.VMEM_SHARED((NROW, 16), jnp.float32),
            pltpu.SemaphoreType.DMA,
        ],
    )(dstd, ones16, zeros16)


def _agg_body(g3, src_hbm, dst_hbm, zeros_hbm, out_hbm,
              sidx_all, didx_all, rows_a1, rows_b1, rows_a2, rows_b2, buf, acc,
              gs_a1, gs_b1, gs_a2, gs_b2, ss_a1, ss_b1, ss_a2, ss_b2):
    c = lax.axis_index("c")
    s = lax.axis_index("s")
    base = s * RPT
    pltpu.sync_copy(zeros_hbm, buf)

    def zero_step(k, _):
        pltpu.sync_copy(buf, acc.at[pl.ds(base + k * C, C)])
        return 0

    lax.fori_loop(0, RPT // C, zero_step, 0)
    plsc.subcore_barrier()

    T = GRP // 2

    def run(g_hbm):
        sets = ((rows_a1, rows_b1, gs_a1, gs_b1, ss_a1, ss_b1),
                (rows_a2, rows_b2, gs_a2, gs_b2, ss_a2, ss_b2))

        def group(g, _):
            pltpu.sync_copy(src_hbm.at[s, pl.ds(g * GRP, GRP)], sidx_all)
            pltpu.sync_copy(dst_hbm.at[s, pl.ds(g * GRP, GRP)], didx_all)
            ra0, rb0, ga0, gb0, _sa0, _sb0 = sets[0]
            pltpu.async_copy(g_hbm.at[sidx_all.at[0]], ra0, ga0)
            pltpu.async_copy(g_hbm.at[sidx_all.at[1]], rb0, gb0)

            def halfstep(t, parity):
                cur = sets[parity]
                nxt = sets[1 - parity]
                ra, rb, ga, gb, sa, sb = cur
                nra, nrb, nga, ngb, nsa, nsb = nxt
                ja = 2 * t
                jb = 2 * t + 1

                @pl.when(t > 0)
                def _():
                    pltpu.make_async_copy(nra, acc.at[didx_all.at[ja]], nsa).wait()
                    pltpu.make_async_copy(nrb, acc.at[didx_all.at[jb]], nsb).wait()

                @pl.when(t < T - 1)
                def _():
                    pltpu.async_copy(g_hbm.at[sidx_all.at[ja + 2]], nra, nga)
                    pltpu.async_copy(g_hbm.at[sidx_all.at[jb + 2]], nrb, ngb)

                pltpu.make_async_copy(g_hbm.at[sidx_all.at[ja]], ra, ga).wait()
                pltpu.async_copy(ra, acc.at[didx_all.at[ja]], sa, add=True)
                pltpu.make_async_copy(g_hbm.at[sidx_all.at[jb]], rb, gb).wait()
                pltpu.async_copy(rb, acc.at[didx_all.at[jb]], sb, add=True)
                return 0

            def step2(u, _):
                halfstep(2 * u, 0)
                halfstep(2 * u + 1, 1)
                return 0

            lax.fori_loop(0, T // 2, step2, 0)
            lra, lrb, _lga, _lgb, lsa, lsb = sets[(T - 1) % 2]
            pltpu.make_async_copy(lra, acc.at[didx_all.at[0]], lsa).wait()
            pltpu.make_async_copy(lrb, acc.at[didx_all.at[1]], lsb).wait()
            return 0

        lax.fori_loop(0, CJ_AGG // GRP, group, 0)

    pl.when(c == 0)(lambda: run(g3.at[0]))
    pl.when(c == 1)(lambda: run(g3.at[1]))
    plsc.subcore_barrier()

    def out_step(k, _):
        pltpu.sync_copy(acc.at[pl.ds(base + k * C, C)], rows_a1)
        pltpu.sync_copy(rows_a1, out_hbm.at[pl.ds(c * NROW + base + k * C, C)])
        return 0

    lax.fori_loop(0, RPT // C, out_step, 0)


def _agg_call(g3, srcp, dstp, zeros32):
    return pl.kernel(
        _agg_body,
        out_type=jax.ShapeDtypeStruct((NC * NROW, HW), jnp.float32),
        mesh=_mesh,
        compiler_params=_sc_params,
        scratch_types=[
            pltpu.VMEM((GRP, C), jnp.int32),
            pltpu.VMEM((GRP, C), jnp.int32),
            pltpu.VMEM((C, HW), jnp.float32),
            pltpu.VMEM((C, HW), jnp.float32),
            pltpu.VMEM((C, HW), jnp.float32),
            pltpu.VMEM((C, HW), jnp.float32),
            pltpu.VMEM((C, HW), jnp.float32),
            pltpu.VMEM_SHARED((NROW, HW), jnp.float32),
            pltpu.SemaphoreType.DMA,
            pltpu.SemaphoreType.DMA,
            pltpu.SemaphoreType.DMA,
            pltpu.SemaphoreType.DMA,
            pltpu.SemaphoreType.DMA,
            pltpu.SemaphoreType.DMA,
            pltpu.SemaphoreType.DMA,
            pltpu.SemaphoreType.DMA,
        ],
    )(g3, srcp, dstp, zeros32)


def _dec_body(z_hbm, es_hbm, ed_hbm, sout_hbm, dout_hbm,
              sidx_all, didx_all, srows, drows, sem_s, sem_d):
    c = lax.axis_index("c")
    s = lax.axis_index("s")
    w = c * NT + s
    pltpu.sync_copy(es_hbm.at[w], sidx_all)
    pltpu.sync_copy(ed_hbm.at[w], didx_all)

    def step(j, _):
        off = (w * CJ_DEC + j) * C
        pltpu.async_copy(z_hbm.at[sidx_all.at[j]], srows, sem_s)
        pltpu.async_copy(z_hbm.at[didx_all.at[j]], drows, sem_d)
        pltpu.make_async_copy(z_hbm.at[sidx_all.at[j]], srows, sem_s).wait()
        pltpu.sync_copy(srows, sout_hbm.at[pl.ds(off, C)])
        pltpu.make_async_copy(z_hbm.at[didx_all.at[j]], drows, sem_d).wait()
        pltpu.sync_copy(drows, dout_hbm.at[pl.ds(off, C)])
        return 0

    lax.fori_loop(0, CJ_DEC, step, 0)


def _dec_call(z2, es, ed):
    return pl.kernel(
        _dec_body,
        out_type=(
            jax.ShapeDtypeStruct((L_PAD, D), jnp.float32),
            jax.ShapeDtypeStruct((L_PAD, D), jnp.float32),
        ),
        mesh=_mesh,
        compiler_params=_sc_params,
        scratch_types=[
            pltpu.VMEM((CJ_DEC, C), jnp.int32),
            pltpu.VMEM((CJ_DEC, C), jnp.int32),
            pltpu.VMEM((C, D), jnp.float32),
            pltpu.VMEM((C, D), jnp.float32),
            pltpu.SemaphoreType.DMA,
            pltpu.SemaphoreType.DMA,
        ],
    )(z2, es, ed)


def _dinv_block(deg_ref):
    dsum = deg_ref[0][:, 0:1] + deg_ref[1][:, 0:1]
    return lax.rsqrt(dsum + 1.0)


def _prep1_body(deg_ref, emb_ref, w1_ref, outg_ref, outh_ref):
    dinv = _dinv_block(deg_ref)
    h = jnp.dot(emb_ref[...], w1_ref[...], preferred_element_type=jnp.float32)
    outh_ref[...] = h
    g = h * dinv
    outg_ref[0] = g[:, :HW]
    outg_ref[1] = g[:, HW:]


def _mid_body(deg_ref, acc_ref, h_ref, b_ref, w2_ref, outg_ref, outh_ref):
    dinv = _dinv_block(deg_ref)
    accf = jnp.concatenate([acc_ref[0], acc_ref[1]], axis=1)
    z = jnp.maximum(accf * dinv + h_ref[...] * (dinv * dinv) + b_ref[...], 0.0)
    h2 = jnp.dot(z, w2_ref[...], preferred_element_type=jnp.float32)
    outh_ref[...] = h2
    g2 = h2 * dinv
    outg_ref[0] = g2[:, :HW]
    outg_ref[1] = g2[:, HW:]


def _fin_body(deg_ref, acc_ref, h_ref, b_ref, out_ref):
    dinv = _dinv_block(deg_ref)
    accf = jnp.concatenate([acc_ref[0], acc_ref[1]], axis=1)
    out_ref[...] = accf * dinv + h_ref[...] * (dinv * dinv) + b_ref[...]


def _dot_body(s_ref, d_ref, o_ref):
    o_ref[...] = jnp.sum(s_ref[...] * d_ref[...], axis=1).reshape(BRL // 256, 256)


_deg_spec = pl.BlockSpec((NC, BR, 16), lambda i: (0, i, 0))
_acc_spec = pl.BlockSpec((NC, BR, HW), lambda i: (0, i, 0))
_row_spec = pl.BlockSpec((BR, D), lambda i: (i, 0))
_g_spec = pl.BlockSpec((NC, BR, HW), lambda i: (0, i, 0))
_w_spec = pl.BlockSpec((D, D), lambda i: (0, 0))
_b_spec = pl.BlockSpec((1, D), lambda i: (0, 0))


def _prep1(deg3, emb, W1):
    return pl.pallas_call(
        _prep1_body,
        grid=(N // BR,),
        in_specs=[_deg_spec, _row_spec, _w_spec],
        out_specs=(_g_spec, _row_spec),
        out_shape=(
            jax.ShapeDtypeStruct((NC, N, HW), jnp.float32),
            jax.ShapeDtypeStruct((N, D), jnp.float32),
        ),
    )(deg3, emb, W1)


def _mid(deg3, acc3, h1, b1, W2):
    return pl.pallas_call(
        _mid_body,
        grid=(N // BR,),
        in_specs=[_deg_spec, _acc_spec, _row_spec, _b_spec, _w_spec],
        out_specs=(_g_spec, _row_spec),
        out_shape=(
            jax.ShapeDtypeStruct((NC, N, HW), jnp.float32),
            jax.ShapeDtypeStruct((N, D), jnp.float32),
        ),
    )(deg3, acc3, h1, b1, W2)


def _fin(deg3, acc3, h2, b2):
    return pl.pallas_call(
        _fin_body,
        grid=(N // BR,),
        in_specs=[_deg_spec, _acc_spec, _row_spec, _b_spec],
        out_specs=_row_spec,
        out_shape=jax.ShapeDtypeStruct((N, D), jnp.float32),
    )(deg3, acc3, h2, b2)


def _dot(srows, drows):
    return pl.pallas_call(
        _dot_body,
        grid=(L_PAD // BRL,),
        in_specs=[
            pl.BlockSpec((BRL, D), lambda i: (i, 0)),
            pl.BlockSpec((BRL, D), lambda i: (i, 0)),
        ],
        out_specs=pl.BlockSpec((BRL // 256, 256), lambda i: (i, 0)),
        out_shape=jax.ShapeDtypeStruct((L_PAD // 256, 256), jnp.float32),
    )(srows, drows)


def kernel(edge_index, edge_label_index, emb, W1, b1, W2, b2):
    src = edge_index[0]
    dst = edge_index[1]

    ones16 = jnp.ones((C, 16), jnp.float32)
    zeros16 = jnp.zeros((C, 16), jnp.float32)
    zeros32 = jnp.zeros((C, HW), jnp.float32)

    dstd = jnp.concatenate(
        [dst, jnp.full((E_DEG - E,), TRASH, jnp.int32)]
    ).reshape(NC * NT, CJ_DEG, C)
    deg3 = _deg_call(dstd, ones16, zeros16).reshape(NC, NROW, 16)

    g1, h1 = _prep1(deg3, emb, W1)

    srcp = jnp.concatenate(
        [src, jnp.zeros((E_AGG - E,), jnp.int32)]
    ).reshape(NT, CJ_AGG, C)
    dstp = jnp.concatenate(
        [dst, jnp.full((E_AGG - E,), TRASH, jnp.int32)]
    ).reshape(NT, CJ_AGG, C)

    acc1 = _agg_call(g1, srcp, dstp, zeros32).reshape(NC, NROW, HW)
    g2, h2 = _mid(deg3, acc1, h1, b1.reshape(1, D), W2)
    acc2 = _agg_call(g2, srcp, dstp, zeros32).reshape(NC, NROW, HW)
    z2 = _fin(deg3, acc2, h2, b2.reshape(1, D))

    es = jnp.concatenate(
        [edge_label_index[0], jnp.zeros((L_PAD - L,), jnp.int32)]
    ).reshape(NC * NT, CJ_DEC, C)
    ed = jnp.concatenate(
        [edge_label_index[1], jnp.zeros((L_PAD - L,), jnp.int32)]
    ).reshape(NC * NT, CJ_DEC, C)
    srows, drows = _dec_call(z2, es, ed)
    return _dot(srows, drows).reshape(L_PAD)[:L]
